# baseline (speedup 1.0000x reference)
import functools

import numpy as np

import jax
import jax.numpy as jnp
from jax import lax
from jax.experimental import pallas as pl
from jax.experimental.pallas import tpu as pltpu
from jax.experimental.pallas import tpu_sc as plsc

E = 8
D_MODEL = 1024
D_OUT = 1024
S = 2048
CAP = 2 * S // E
NSLOT = E * CAP
ZROW = NSLOT
NPAD = 4608
STT = NSLOT + 16

NC, NS = 2, 16
NW = NC * NS
SLOTS_PER_W = NSLOT // NW
TOKS_PER_W = S // NW


def _gating_body(x_ref, wg_ref, gn_ref, ts1_ref, ts2_ref, g1_ref,
                 g2_ref, xi_ref):
    x = x_ref[...]
    wg = wg_ref[...]
    logits = lax.dot_general(x, wg, (((1,), (1,)), ((), ())),
                             preferred_element_type=jnp.float32)
    lanes = lax.broadcasted_iota(jnp.int32, (S, E), 1)

    m = jnp.max(logits, axis=1, keepdims=True)
    ex = jnp.exp(logits - m)
    gates = ex / jnp.sum(ex, axis=1, keepdims=True)

    m1 = jnp.max(logits, axis=1, keepdims=True)
    idx1 = jnp.min(jnp.where(logits == m1, lanes, E), axis=1, keepdims=True)
    mask1 = lanes == idx1

    lwn = logits + gn_ref[...]
    l2 = jnp.where(mask1, -jnp.inf, lwn)
    m2 = jnp.max(l2, axis=1, keepdims=True)
    idx2 = jnp.min(jnp.where(l2 == m2, lanes, E), axis=1, keepdims=True)
    mask2 = lanes == idx2

    m1f = mask1.astype(jnp.float32)
    m2f = mask2.astype(jnp.float32)

    row = lax.broadcasted_iota(jnp.int32, (S, S), 0)
    col = lax.broadcasted_iota(jnp.int32, (S, S), 1)
    tri = (col <= row).astype(jnp.bfloat16)
    cat = jnp.concatenate([m1f, m2f], axis=1).astype(jnp.bfloat16)
    cums = lax.dot_general(tri, cat, (((1,), (0,)), ((), ())),
                           preferred_element_type=jnp.float32)
    loc1 = cums[:, :E] - 1.0
    n1tot = jnp.sum(m1f, axis=0, keepdims=True)
    loc2 = cums[:, E:] - 1.0 + n1tot

    c1 = jnp.sum(jnp.where(mask1, loc1, 0.0), axis=1, keepdims=True)
    c2 = jnp.sum(jnp.where(mask2, loc2, 0.0), axis=1, keepdims=True)
    v1 = c1 < CAP
    v2 = c2 < CAP

    g1r = jnp.where(v1, jnp.sum(jnp.where(mask1, gates, 0.0), axis=1,
                                keepdims=True), 0.0)
    g2r = jnp.where(v2, jnp.sum(jnp.where(mask2, gates, 0.0), axis=1,
                                keepdims=True), 0.0)
    denom = jnp.maximum(g1r + g2r, jnp.finfo(jnp.float32).eps)
    g1_ref[...] = jnp.reshape(g1r / denom, (S // 128, 128))
    g2_ref[...] = jnp.reshape(g2r / denom, (S // 128, 128))

    ts1 = jnp.where(v1, idx1 * CAP + c1.astype(jnp.int32), ZROW)
    ts2 = jnp.where(v2, idx2 * CAP + c2.astype(jnp.int32), ZROW)
    ts1_ref[...] = jnp.reshape(ts1, (S // 128, 128))
    ts2_ref[...] = jnp.reshape(ts2, (S // 128, 128))

    xpk = pltpu.pack_elementwise(
        [x[:, :D_MODEL // 2], x[:, D_MODEL // 2:]],
        packed_dtype=jnp.bfloat16)
    xi_ref[...] = lax.bitcast_convert_type(xpk, jnp.int32)


_gating_call = pl.pallas_call(
    _gating_body,
    out_shape=[
        jax.ShapeDtypeStruct((S // 128, 128), jnp.int32),
        jax.ShapeDtypeStruct((S // 128, 128), jnp.int32),
        jax.ShapeDtypeStruct((S // 128, 128), jnp.float32),
        jax.ShapeDtypeStruct((S // 128, 128), jnp.float32),
        jax.ShapeDtypeStruct((S, D_MODEL // 2), jnp.int32),
    ],
)


def _dispatch_body(ts1_hbm, ts2_hbm, g1_hbm, g2_hbm, x_hbm,
                   disp_hbm, gs_hbm,
                   ts1_v, ts2_v, g1_v, g2_v, stt_v, gs_v,
                   rows_a, rows_b, sem_a, sem_b):
    cid = lax.axis_index("c")
    sid = lax.axis_index("s")
    wid = sid * NC + cid
    sbase = wid * SLOTS_PER_W

    d1 = pltpu.async_copy(ts1_hbm, ts1_v, sem_a)
    d2 = pltpu.async_copy(ts2_hbm, ts2_v, sem_b)
    d3 = pltpu.async_copy(g1_hbm, g1_v, sem_a)
    d4 = pltpu.async_copy(g2_hbm, g2_v, sem_b)
    d1.wait()
    d2.wait()
    d3.wait()
    d4.wait()

    zi = jnp.zeros((16,), jnp.int32)
    zf = jnp.zeros((16,), jnp.float32)

    def zero_own(i, c):
        stt_v[pl.ds(sbase + i * 16, 16)] = zi
        gs_v[pl.ds(sbase + i * 16, 16)] = zf
        return c
    lax.fori_loop(0, SLOTS_PER_W // 16, zero_own, 0)

    base_ids = lax.iota(jnp.int32, 16)

    def scat_stt(i, c):
        sl = pl.ds(i * 16, 16)
        ids = base_ids + i * 16
        plsc.store_scatter(stt_v, [ts1_v[sl]], ids)
        plsc.store_scatter(stt_v, [ts2_v[sl]], ids)
        return c
    lax.fori_loop(0, S // 16, scat_stt, 0)

    def gather(q, dst, sem):
        sl = stt_v.at[pl.ds(sbase + q * 64, 64)]
        return pltpu.async_copy(x_hbm.at[sl], dst, sem)

    def write(q, src, sem):
        return pltpu.async_copy(src, disp_hbm.at[pl.ds(sbase + q * 64, 64)],
                                sem)

    da = gather(0, rows_a, sem_a)
    db = gather(1, rows_b, sem_b)

    def scat_gs(i, c):
        sl = pl.ds(i * 16, 16)
        plsc.store_scatter(gs_v, [ts1_v[sl]], g1_v[sl])
        plsc.store_scatter(gs_v, [ts2_v[sl]], g2_v[sl])
        return c
    lax.fori_loop(0, S // 16, scat_gs, 0)

    da.wait()
    wa = write(0, rows_a, sem_a)
    db.wait()
    wb = write(1, rows_b, sem_b)

    pltpu.sync_copy(gs_v.at[pl.ds(sbase, SLOTS_PER_W)],
                    gs_hbm.at[pl.ds(sbase, SLOTS_PER_W)])

    @pl.when(wid == 0)
    def _():
        def zero_pad(i, c):
            gs_v[pl.ds(NSLOT + i * 16, 16)] = zf
            return c
        lax.fori_loop(0, (NPAD - NSLOT) // 16, zero_pad, 0)
        pltpu.sync_copy(gs_v.at[pl.ds(NSLOT, NPAD - NSLOT)],
                        gs_hbm.at[pl.ds(NSLOT, NPAD - NSLOT)])

    wa.wait()
    wb.wait()


@functools.cache
def _get_dispatch_call():
  return pl.kernel(
    _dispatch_body,
    out_type=[
        jax.ShapeDtypeStruct((NPAD, D_MODEL // 2), jnp.int32),
        jax.ShapeDtypeStruct((NPAD,), jnp.float32),
    ],
    mesh=plsc.VectorSubcoreMesh(core_axis_name="c", subcore_axis_name="s",
                                num_cores=NC, num_subcores=NS),
    compiler_params=pltpu.CompilerParams(needs_layout_passes=False),
    scratch_types=[
        pltpu.VMEM((S,), jnp.int32),
        pltpu.VMEM((S,), jnp.int32),
        pltpu.VMEM((S,), jnp.float32),
        pltpu.VMEM((S,), jnp.float32),
        pltpu.VMEM((STT,), jnp.int32),
        pltpu.VMEM((NPAD,), jnp.float32),
        pltpu.VMEM((64, D_MODEL // 2), jnp.int32),
        pltpu.VMEM((64, D_MODEL // 2), jnp.int32),
        pltpu.SemaphoreType.DMA,
        pltpu.SemaphoreType.DMA,
    ],
  )


def _wcast_body(w_ref, o_ref):
    o_ref[...] = w_ref[...].astype(jnp.bfloat16)


_wcast_call = pl.pallas_call(
    _wcast_body,
    grid=(2 * E,),
    in_specs=[pl.BlockSpec((1, D_OUT // 2, D_MODEL), lambda i: (i // 2, i % 2, 0))],
    out_specs=pl.BlockSpec((1, D_OUT // 2, D_MODEL), lambda i: (i // 2, i % 2, 0)),
    out_shape=jax.ShapeDtypeStruct((E, D_OUT, D_MODEL), jnp.bfloat16),
)


def _mm_body(d_ref, w_ref, b_ref, s_ref, y_ref):
    xi = d_ref[...]
    lo = pltpu.unpack_elementwise(
        xi, index=0, packed_dtype=jnp.bfloat16,
        unpacked_dtype=jnp.float32).astype(jnp.bfloat16)
    hi = pltpu.unpack_elementwise(
        xi, index=1, packed_dtype=jnp.bfloat16,
        unpacked_dtype=jnp.float32).astype(jnp.bfloat16)
    dims = (((1,), (1,)), ((), ()))
    w = w_ref[0]
    y = lax.dot_general(lo, w[:, :D_MODEL // 2], dims,
                        preferred_element_type=jnp.float32)
    y += lax.dot_general(hi, w[:, D_MODEL // 2:], dims,
                         preferred_element_type=jnp.float32)
    y = y + b_ref[0]
    s = jnp.reshape(s_ref[...], (CAP, 1))
    y = jnp.where(s > 0.0, y * s, 0.0)
    ypk = pltpu.pack_elementwise(
        [y[:, :D_OUT // 2], y[:, D_OUT // 2:]], packed_dtype=jnp.bfloat16)
    y_ref[...] = lax.bitcast_convert_type(ypk, jnp.int32)


_mm_call = pl.pallas_call(
    _mm_body,
    grid=(NPAD // CAP,),
    in_specs=[
        pl.BlockSpec((CAP, D_MODEL // 2), lambda i: (i, 0)),
        pl.BlockSpec((1, D_OUT, D_MODEL),
                     lambda i: (jnp.minimum(i, E - 1), 0, 0)),
        pl.BlockSpec((1, 1, D_OUT), lambda i: (jnp.minimum(i, E - 1), 0, 0)),
        pl.BlockSpec((CAP,), lambda i: (i,)),
    ],
    out_specs=pl.BlockSpec((CAP, D_OUT // 2), lambda i: (i, 0)),
    out_shape=jax.ShapeDtypeStruct((NPAD, D_OUT // 2), jnp.int32),
)


_CCH = 16


def _combine_body(ts1_hbm, ts2_hbm, y_hbm, out_hbm,
                  ts1_v, ts2_v, a1, a2, b1, b2, oa, ob,
                  sa1, sa2, sb1, sb2, swa, swb):
    cid = lax.axis_index("c")
    sid = lax.axis_index("s")
    wid = sid * NC + cid
    base = wid * TOKS_PER_W

    pltpu.sync_copy(ts1_hbm.at[pl.ds(base, TOKS_PER_W)], ts1_v)
    pltpu.sync_copy(ts2_hbm.at[pl.ds(base, TOKS_PER_W)], ts2_v)

    def gather(c, d1, d2, s1, s2):
        g1 = pltpu.async_copy(y_hbm.at[ts1_v.at[pl.ds(c * _CCH, _CCH)]], d1, s1)
        g2 = pltpu.async_copy(y_hbm.at[ts2_v.at[pl.ds(c * _CCH, _CCH)]], d2, s2)
        return g1, g2

    def add_unpack(d1, d2, o):
        def addrow(r, c):
            @plsc.parallel_loop(0, D_OUT // 2, 16, unroll=8)
            def _(j):
                af = plsc.bitcast(d1[r, pl.ds(j, 16)], jnp.bfloat16)
                bf = plsc.bitcast(d2[r, pl.ds(j, 16)], jnp.bfloat16)
                lo, hi = plsc.unpack(af + bf,
                                     format=plsc.PackFormat.INTERLEAVED)
                o[r, pl.ds(j, 16)] = lo
                o[r, pl.ds(D_OUT // 2 + j, 16)] = hi
            return c
        lax.fori_loop(0, _CCH, addrow, 0)

    def write(o, c, sem):
        return pltpu.async_copy(o, out_hbm.at[pl.ds(base + c * _CCH, _CCH)],
                                sem)

    ga = gather(0, a1, a2, sa1, sa2)
    gb = gather(1, b1, b2, sb1, sb2)

    ga[0].wait(); ga[1].wait()
    add_unpack(a1, a2, oa)
    ga = gather(2, a1, a2, sa1, sa2)
    wa = write(oa, 0, swa)

    gb[0].wait(); gb[1].wait()
    add_unpack(b1, b2, ob)
    gb = gather(3, b1, b2, sb1, sb2)
    wb = write(ob, 1, swb)

    ga[0].wait(); ga[1].wait()
    wa.wait()
    add_unpack(a1, a2, oa)
    wa = write(oa, 2, swa)

    gb[0].wait(); gb[1].wait()
    wb.wait()
    add_unpack(b1, b2, ob)
    wb = write(ob, 3, swb)

    wa.wait()
    wb.wait()


@functools.cache
def _get_combine_call():
  return pl.kernel(
    _combine_body,
    out_type=jax.ShapeDtypeStruct((S, D_OUT), jnp.float32),
    mesh=plsc.VectorSubcoreMesh(core_axis_name="c", subcore_axis_name="s",
                                num_cores=NC, num_subcores=NS),
    compiler_params=pltpu.CompilerParams(needs_layout_passes=False),
    scratch_types=[
        pltpu.VMEM((TOKS_PER_W,), jnp.int32),
        pltpu.VMEM((TOKS_PER_W,), jnp.int32),
        pltpu.VMEM((_CCH, D_OUT // 2), jnp.int32),
        pltpu.VMEM((_CCH, D_OUT // 2), jnp.int32),
        pltpu.VMEM((_CCH, D_OUT // 2), jnp.int32),
        pltpu.VMEM((_CCH, D_OUT // 2), jnp.int32),
        pltpu.VMEM((_CCH, D_OUT), jnp.float32),
        pltpu.VMEM((_CCH, D_OUT), jnp.float32),
        pltpu.SemaphoreType.DMA,
        pltpu.SemaphoreType.DMA,
        pltpu.SemaphoreType.DMA,
        pltpu.SemaphoreType.DMA,
        pltpu.SemaphoreType.DMA,
        pltpu.SemaphoreType.DMA,
    ],
  )


@jax.jit
def kernel(x, wg, We, be, gnoise):
    x2 = x.reshape(-1, x.shape[-1])
    ts1, ts2, g1, g2, xi = _gating_call(x2, wg, gnoise)
    ts1 = ts1.reshape(-1)
    ts2 = ts2.reshape(-1)
    wb = _wcast_call(We)
    disp, gs = _get_dispatch_call()(ts1, ts2, g1.reshape(-1), g2.reshape(-1),
                                    xi)
    y = _mm_call(disp, wb, be.reshape(E, 1, D_OUT), gs)
    out = _get_combine_call()(ts1, ts2, y)
    return out.reshape(x.shape)


# speedup vs baseline: 1.2014x; 1.2014x over previous
import functools

import numpy as np

import jax
import jax.numpy as jnp
from jax import lax
from jax.experimental import pallas as pl
from jax.experimental.pallas import tpu as pltpu
from jax.experimental.pallas import tpu_sc as plsc

E = 8
D_MODEL = 1024
D_OUT = 1024
S = 2048
CAP = 2 * S // E
NSLOT = E * CAP
ZROW = NSLOT
NPAD = 4608
STT = NSLOT + 16

NC, NS = 2, 16
NW = NC * NS
SLOTS_PER_W = NSLOT // NW
TOKS_PER_W = S // NW


def _gating_body(x_ref, wg_ref, gn_ref, ts1_ref, ts2_ref, g1_ref,
                 g2_ref, xi_ref):
    x = x_ref[...]
    wg = wg_ref[...]
    logits = lax.dot_general(x, wg, (((1,), (1,)), ((), ())),
                             preferred_element_type=jnp.float32)
    lanes = lax.broadcasted_iota(jnp.int32, (S, E), 1)

    m = jnp.max(logits, axis=1, keepdims=True)
    ex = jnp.exp(logits - m)
    gates = ex / jnp.sum(ex, axis=1, keepdims=True)

    m1 = jnp.max(logits, axis=1, keepdims=True)
    idx1 = jnp.min(jnp.where(logits == m1, lanes, E), axis=1, keepdims=True)
    mask1 = lanes == idx1

    lwn = logits + gn_ref[...]
    l2 = jnp.where(mask1, -jnp.inf, lwn)
    m2 = jnp.max(l2, axis=1, keepdims=True)
    idx2 = jnp.min(jnp.where(l2 == m2, lanes, E), axis=1, keepdims=True)
    mask2 = lanes == idx2

    m1f = mask1.astype(jnp.float32)
    m2f = mask2.astype(jnp.float32)

    cums = jnp.concatenate([m1f, m2f], axis=1)
    k = 1
    while k < S:
        z = jnp.zeros((k, 2 * E), jnp.float32)
        cums = cums + jnp.concatenate([z, cums[:-k]], axis=0)
        k *= 2
    loc1 = cums[:, :E] - 1.0
    n1tot = jnp.sum(m1f, axis=0, keepdims=True)
    loc2 = cums[:, E:] - 1.0 + n1tot

    c1 = jnp.sum(jnp.where(mask1, loc1, 0.0), axis=1, keepdims=True)
    c2 = jnp.sum(jnp.where(mask2, loc2, 0.0), axis=1, keepdims=True)
    v1 = c1 < CAP
    v2 = c2 < CAP

    g1r = jnp.where(v1, jnp.sum(jnp.where(mask1, gates, 0.0), axis=1,
                                keepdims=True), 0.0)
    g2r = jnp.where(v2, jnp.sum(jnp.where(mask2, gates, 0.0), axis=1,
                                keepdims=True), 0.0)
    denom = jnp.maximum(g1r + g2r, jnp.finfo(jnp.float32).eps)
    g1_ref[...] = jnp.reshape(g1r / denom, (S // 128, 128))
    g2_ref[...] = jnp.reshape(g2r / denom, (S // 128, 128))

    ts1 = jnp.where(v1, idx1 * CAP + c1.astype(jnp.int32), ZROW)
    ts2 = jnp.where(v2, idx2 * CAP + c2.astype(jnp.int32), ZROW)
    ts1_ref[...] = jnp.reshape(ts1, (S // 128, 128))
    ts2_ref[...] = jnp.reshape(ts2, (S // 128, 128))

    xpk = pltpu.pack_elementwise(
        [x[:, :D_MODEL // 2], x[:, D_MODEL // 2:]],
        packed_dtype=jnp.bfloat16)
    xi_ref[...] = lax.bitcast_convert_type(xpk, jnp.int32)


_gating_call = pl.pallas_call(
    _gating_body,
    out_shape=[
        jax.ShapeDtypeStruct((S // 128, 128), jnp.int32),
        jax.ShapeDtypeStruct((S // 128, 128), jnp.int32),
        jax.ShapeDtypeStruct((S // 128, 128), jnp.float32),
        jax.ShapeDtypeStruct((S // 128, 128), jnp.float32),
        jax.ShapeDtypeStruct((S, D_MODEL // 2), jnp.int32),
    ],
)


def _dispatch_body(ts1_hbm, ts2_hbm, g1_hbm, g2_hbm, x_hbm,
                   disp_hbm, gs_hbm,
                   ts1_v, ts2_v, g1_v, g2_v, stt_v, gs_v,
                   rows_a, rows_b, sem_a, sem_b):
    cid = lax.axis_index("c")
    sid = lax.axis_index("s")
    wid = sid * NC + cid
    sbase = wid * SLOTS_PER_W

    d1 = pltpu.async_copy(ts1_hbm, ts1_v, sem_a)
    d2 = pltpu.async_copy(ts2_hbm, ts2_v, sem_b)
    d3 = pltpu.async_copy(g1_hbm, g1_v, sem_a)
    d4 = pltpu.async_copy(g2_hbm, g2_v, sem_b)
    d1.wait()
    d2.wait()
    d3.wait()
    d4.wait()

    zi = jnp.zeros((16,), jnp.int32)
    zf = jnp.zeros((16,), jnp.float32)

    def zero_own(i, c):
        stt_v[pl.ds(sbase + i * 16, 16)] = zi
        gs_v[pl.ds(sbase + i * 16, 16)] = zf
        return c
    lax.fori_loop(0, SLOTS_PER_W // 16, zero_own, 0)

    base_ids = lax.iota(jnp.int32, 16)

    def scat_stt(i, c):
        sl = pl.ds(i * 16, 16)
        ids = base_ids + i * 16
        plsc.store_scatter(stt_v, [ts1_v[sl]], ids)
        plsc.store_scatter(stt_v, [ts2_v[sl]], ids)
        return c
    lax.fori_loop(0, S // 16, scat_stt, 0)

    def gather(q, dst, sem):
        sl = stt_v.at[pl.ds(sbase + q * 64, 64)]
        return pltpu.async_copy(x_hbm.at[sl], dst, sem)

    def write(q, src, sem):
        return pltpu.async_copy(src, disp_hbm.at[pl.ds(sbase + q * 64, 64)],
                                sem)

    da = gather(0, rows_a, sem_a)
    db = gather(1, rows_b, sem_b)

    def scat_gs(i, c):
        sl = pl.ds(i * 16, 16)
        plsc.store_scatter(gs_v, [ts1_v[sl]], g1_v[sl])
        plsc.store_scatter(gs_v, [ts2_v[sl]], g2_v[sl])
        return c
    lax.fori_loop(0, S // 16, scat_gs, 0)

    da.wait()
    wa = write(0, rows_a, sem_a)
    db.wait()
    wb = write(1, rows_b, sem_b)

    pltpu.sync_copy(gs_v.at[pl.ds(sbase, SLOTS_PER_W)],
                    gs_hbm.at[pl.ds(sbase, SLOTS_PER_W)])

    @pl.when(wid == 0)
    def _():
        def zero_pad(i, c):
            gs_v[pl.ds(NSLOT + i * 16, 16)] = zf
            return c
        lax.fori_loop(0, (NPAD - NSLOT) // 16, zero_pad, 0)
        pltpu.sync_copy(gs_v.at[pl.ds(NSLOT, NPAD - NSLOT)],
                        gs_hbm.at[pl.ds(NSLOT, NPAD - NSLOT)])

    wa.wait()
    wb.wait()


@functools.cache
def _get_dispatch_call():
  return pl.kernel(
    _dispatch_body,
    out_type=[
        jax.ShapeDtypeStruct((NPAD, D_MODEL // 2), jnp.int32),
        jax.ShapeDtypeStruct((NPAD,), jnp.float32),
    ],
    mesh=plsc.VectorSubcoreMesh(core_axis_name="c", subcore_axis_name="s",
                                num_cores=NC, num_subcores=NS),
    compiler_params=pltpu.CompilerParams(needs_layout_passes=False),
    scratch_types=[
        pltpu.VMEM((S,), jnp.int32),
        pltpu.VMEM((S,), jnp.int32),
        pltpu.VMEM((S,), jnp.float32),
        pltpu.VMEM((S,), jnp.float32),
        pltpu.VMEM((STT,), jnp.int32),
        pltpu.VMEM((NPAD,), jnp.float32),
        pltpu.VMEM((64, D_MODEL // 2), jnp.int32),
        pltpu.VMEM((64, D_MODEL // 2), jnp.int32),
        pltpu.SemaphoreType.DMA,
        pltpu.SemaphoreType.DMA,
    ],
  )


def _mm_body(d_ref, w_ref, b_ref, s_ref, y_ref):
    xi = d_ref[...]
    lo = pltpu.unpack_elementwise(
        xi, index=0, packed_dtype=jnp.bfloat16,
        unpacked_dtype=jnp.float32).astype(jnp.bfloat16)
    hi = pltpu.unpack_elementwise(
        xi, index=1, packed_dtype=jnp.bfloat16,
        unpacked_dtype=jnp.float32).astype(jnp.bfloat16)
    dims = (((1,), (1,)), ((), ()))
    w = w_ref[0].astype(jnp.bfloat16)
    y = lax.dot_general(lo, w[:, :D_MODEL // 2], dims,
                        preferred_element_type=jnp.float32)
    y += lax.dot_general(hi, w[:, D_MODEL // 2:], dims,
                         preferred_element_type=jnp.float32)
    y = y + b_ref[0]
    s = jnp.reshape(s_ref[...], (CAP, 1))
    y = jnp.where(s > 0.0, y * s, 0.0)
    ypk = pltpu.pack_elementwise(
        [y[:, :D_OUT // 2], y[:, D_OUT // 2:]], packed_dtype=jnp.bfloat16)
    y_ref[...] = lax.bitcast_convert_type(ypk, jnp.int32)


_mm_call = pl.pallas_call(
    _mm_body,
    grid=(NPAD // CAP,),
    in_specs=[
        pl.BlockSpec((CAP, D_MODEL // 2), lambda i: (i, 0)),
        pl.BlockSpec((1, D_OUT, D_MODEL),
                     lambda i: (jnp.minimum(i, E - 1), 0, 0)),
        pl.BlockSpec((1, 1, D_OUT), lambda i: (jnp.minimum(i, E - 1), 0, 0)),
        pl.BlockSpec((CAP,), lambda i: (i,)),
    ],
    out_specs=pl.BlockSpec((CAP, D_OUT // 2), lambda i: (i, 0)),
    out_shape=jax.ShapeDtypeStruct((NPAD, D_OUT // 2), jnp.int32),
)


_CCH = 16


def _combine_body(ts1_hbm, ts2_hbm, y_hbm, out_hbm,
                  ts1_v, ts2_v, a1, a2, b1, b2, oa, ob,
                  sa1, sa2, sb1, sb2, swa, swb):
    cid = lax.axis_index("c")
    sid = lax.axis_index("s")
    wid = sid * NC + cid
    base = wid * TOKS_PER_W

    pltpu.sync_copy(ts1_hbm.at[pl.ds(base, TOKS_PER_W)], ts1_v)
    pltpu.sync_copy(ts2_hbm.at[pl.ds(base, TOKS_PER_W)], ts2_v)

    def gather(c, d1, d2, s1, s2):
        g1 = pltpu.async_copy(y_hbm.at[ts1_v.at[pl.ds(c * _CCH, _CCH)]], d1, s1)
        g2 = pltpu.async_copy(y_hbm.at[ts2_v.at[pl.ds(c * _CCH, _CCH)]], d2, s2)
        return g1, g2

    def add_unpack(d1, d2, o):
        def addrow(r, c):
            @plsc.parallel_loop(0, D_OUT // 2, 16, unroll=8)
            def _(j):
                af = plsc.bitcast(d1[r, pl.ds(j, 16)], jnp.bfloat16)
                bf = plsc.bitcast(d2[r, pl.ds(j, 16)], jnp.bfloat16)
                lo, hi = plsc.unpack(af + bf,
                                     format=plsc.PackFormat.INTERLEAVED)
                o[r, pl.ds(j, 16)] = lo
                o[r, pl.ds(D_OUT // 2 + j, 16)] = hi
            return c
        lax.fori_loop(0, _CCH, addrow, 0)

    def write(o, c, sem):
        return pltpu.async_copy(o, out_hbm.at[pl.ds(base + c * _CCH, _CCH)],
                                sem)

    ga = gather(0, a1, a2, sa1, sa2)
    gb = gather(1, b1, b2, sb1, sb2)

    ga[0].wait(); ga[1].wait()
    add_unpack(a1, a2, oa)
    ga = gather(2, a1, a2, sa1, sa2)
    wa = write(oa, 0, swa)

    gb[0].wait(); gb[1].wait()
    add_unpack(b1, b2, ob)
    gb = gather(3, b1, b2, sb1, sb2)
    wb = write(ob, 1, swb)

    ga[0].wait(); ga[1].wait()
    wa.wait()
    add_unpack(a1, a2, oa)
    wa = write(oa, 2, swa)

    gb[0].wait(); gb[1].wait()
    wb.wait()
    add_unpack(b1, b2, ob)
    wb = write(ob, 3, swb)

    wa.wait()
    wb.wait()


@functools.cache
def _get_combine_call():
  return pl.kernel(
    _combine_body,
    out_type=jax.ShapeDtypeStruct((S, D_OUT), jnp.float32),
    mesh=plsc.VectorSubcoreMesh(core_axis_name="c", subcore_axis_name="s",
                                num_cores=NC, num_subcores=NS),
    compiler_params=pltpu.CompilerParams(needs_layout_passes=False),
    scratch_types=[
        pltpu.VMEM((TOKS_PER_W,), jnp.int32),
        pltpu.VMEM((TOKS_PER_W,), jnp.int32),
        pltpu.VMEM((_CCH, D_OUT // 2), jnp.int32),
        pltpu.VMEM((_CCH, D_OUT // 2), jnp.int32),
        pltpu.VMEM((_CCH, D_OUT // 2), jnp.int32),
        pltpu.VMEM((_CCH, D_OUT // 2), jnp.int32),
        pltpu.VMEM((_CCH, D_OUT), jnp.float32),
        pltpu.VMEM((_CCH, D_OUT), jnp.float32),
        pltpu.SemaphoreType.DMA,
        pltpu.SemaphoreType.DMA,
        pltpu.SemaphoreType.DMA,
        pltpu.SemaphoreType.DMA,
        pltpu.SemaphoreType.DMA,
        pltpu.SemaphoreType.DMA,
    ],
  )


@jax.jit
def kernel(x, wg, We, be, gnoise):
    x2 = x.reshape(-1, x.shape[-1])
    ts1, ts2, g1, g2, xi = _gating_call(x2, wg, gnoise)
    ts1 = ts1.reshape(-1)
    ts2 = ts2.reshape(-1)
    disp, gs = _get_dispatch_call()(ts1, ts2, g1.reshape(-1), g2.reshape(-1),
                                    xi)
    y = _mm_call(disp, We, be.reshape(E, 1, D_OUT), gs)
    out = _get_combine_call()(ts1, ts2, y)
    return out.reshape(x.shape)
